# baseline (speedup 1.0000x reference)
import functools

import jax
import jax.numpy as jnp
from jax import lax
from jax.experimental import pallas as pl
from jax.experimental.pallas import tpu as pltpu
from jax.experimental.pallas import tpu_sc as plsc

_HID = 128
_NJ = _HID // 16
_C = 128
_NW = 32
_NBUF = 3
_G = 16

_GDN = lax.GatherDimensionNumbers(
    offset_dims=(), collapsed_slice_dims=(0,), start_index_map=(0,))


def _lane_allreduce(v):
  for k in (8, 4, 2, 1):
    perm = (jnp.arange(16, dtype=jnp.int32) ^ k)[:, None]
    v = v + lax.gather(v, perm, _GDN, (1,),
                       mode=lax.GatherScatterMode.PROMISE_IN_BOUNDS)
  return v


def _ln_token(rows_b, comb_v, gs, bs, t, cix):
  x = [rows_b[t, pl.ds(16 * j, 16)] + comb_v[cix, pl.ds(16 * j, 16)]
       for j in range(_NJ)]
  s = ((x[0] + x[1]) + (x[2] + x[3])) + ((x[4] + x[5]) + (x[6] + x[7]))
  q = [v * v for v in x]
  ss = ((q[0] + q[1]) + (q[2] + q[3])) + ((q[4] + q[5]) + (q[6] + q[7]))
  mean_v = _lane_allreduce(s) * (1.0 / _HID)
  a_v = _lane_allreduce(ss) * (1.0 / _HID) - mean_v * mean_v + 1e-12
  ii = lax.bitcast_convert_type(a_v, jnp.int32)
  ii = jnp.int32(0x5F3759DF) - lax.shift_right_logical(ii, 1)
  y = lax.bitcast_convert_type(ii, jnp.float32)
  y = y * (1.5 - 0.5 * a_v * y * y)
  y = y * (1.5 - 0.5 * a_v * y * y)
  for j in range(_NJ):
    rows_b[t, pl.ds(16 * j, 16)] = (x[j] - mean_v) * y * gs[j] + bs[j]


def _embed_ln(tokens_flat, segments_flat, word_emb, comb, gamma, beta, L):
  N = tokens_flat.shape[0]
  n_chunks = N // _C
  per_w = n_chunks // _NW
  npw = per_w * _C

  mesh = plsc.VectorSubcoreMesh(core_axis_name="c", subcore_axis_name="s")

  @functools.partial(
      pl.kernel,
      out_type=jax.ShapeDtypeStruct((N, _HID), jnp.float32),
      mesh=mesh,
      scratch_types=[
          pltpu.VMEM((npw,), jnp.int32),
          pltpu.VMEM((npw,), jnp.int32),
          pltpu.VMEM((npw + 16,), jnp.int32),
          pltpu.VMEM((_NBUF, _C, _HID), jnp.float32),
          pltpu.VMEM((2 * L, _HID), jnp.float32),
          pltpu.VMEM((2, _HID), jnp.float32),
          pltpu.SemaphoreType.DMA((_NBUF,)),
          pltpu.SemaphoreType.DMA((_NBUF,)),
      ],
  )
  def body(tok_hbm, seg_hbm, word_hbm, comb_hbm, gamma_hbm, beta_hbm, out_hbm,
           tok_v, seg_v, cix_v, rows_v, comb_v, gb_v, in_sem, out_sem):
    wid = lax.axis_index("s") * 2 + lax.axis_index("c")
    w0 = wid * npw
    pltpu.sync_copy(gamma_hbm, gb_v.at[0])
    pltpu.sync_copy(beta_hbm, gb_v.at[1])
    pltpu.sync_copy(comb_hbm, comb_v)
    pltpu.sync_copy(tok_hbm.at[pl.ds(w0, npw)], tok_v)
    pltpu.sync_copy(seg_hbm.at[pl.ds(w0, npw)], seg_v)
    iota = lax.iota(jnp.int32, 16)

    def cix_group(g, c2):
      pos = lax.rem(w0 + g * 16 + iota, L)
      cix_v[pl.ds(g * 16, 16)] = 2 * pos + seg_v[pl.ds(g * 16, 16)]
      return c2

    lax.fori_loop(0, npw // 16, cix_group, 0)

    def issue(c):
      b = lax.rem(c, _NBUF)
      pltpu.async_copy(word_hbm.at[tok_v.at[pl.ds(c * _C, _C)]], rows_v.at[b],
                       in_sem.at[b])

    issue(0)
    issue(1)

    def loop_body(i, carry):
      gs, bs = carry
      b = lax.rem(i, _NBUF)
      base = w0 + i * _C
      pltpu.make_async_copy(word_hbm.at[pl.ds(0, _C)], rows_v.at[b],
                            in_sem.at[b]).wait()
      rows_b = rows_v.at[b]

      @plsc.parallel_loop(0, _C, unroll=8, carry=(gs, bs))
      def tok(t, c2):
        gs2, bs2 = c2
        cix = cix_v[pl.ds(i * _C + t, 16)][0]
        _ln_token(rows_b, comb_v, gs2, bs2, t, cix)
        return c2

      pltpu.async_copy(rows_v.at[b], out_hbm.at[pl.ds(base, _C)],
                       out_sem.at[b])

      @pl.when(i + 2 < per_w)
      def _():
        b2 = lax.rem(i + 2, _NBUF)

        @pl.when(i >= 1)
        def _():
          pltpu.make_async_copy(rows_v.at[b2], out_hbm.at[pl.ds(0, _C)],
                                out_sem.at[b2]).wait()

        issue(i + 2)

      return gs, bs

    gs0 = tuple(gb_v[0, pl.ds(16 * j, 16)] for j in range(_NJ))
    bs0 = tuple(gb_v[1, pl.ds(16 * j, 16)] for j in range(_NJ))
    lax.fori_loop(0, per_w, loop_body, (gs0, bs0))
    for k in range(_NBUF):
      pltpu.make_async_copy(rows_v.at[k], out_hbm.at[pl.ds(0, _C)],
                            out_sem.at[k]).wait()

  return body(tokens_flat, segments_flat, word_emb, comb, gamma, beta)


def kernel(tokens, segments, word_emb, pos_emb, type_emb, gamma, beta):
  B, L = tokens.shape
  comb = (pos_emb[:L, None, :] + type_emb[None, :, :]).reshape(2 * L, _HID)
  out = _embed_ln(tokens.astype(jnp.int32).reshape(-1),
                  segments.astype(jnp.int32).reshape(-1),
                  word_emb, comb, gamma, beta, L)
  return out.reshape(B, L, _HID)


# speedup vs baseline: 1.3898x; 1.3898x over previous
import functools

import jax
import jax.numpy as jnp
from jax import lax
from jax.experimental import pallas as pl
from jax.experimental.pallas import tpu as pltpu
from jax.experimental.pallas import tpu_sc as plsc

_HID = 128
_C = 128
_NW = 32
_NBUF = 3
_BB = 16


def _sc_gather(tokens_flat, word_emb):
  N = tokens_flat.shape[0]
  per_w = N // _C // _NW
  npw = per_w * _C

  mesh = plsc.VectorSubcoreMesh(core_axis_name="c", subcore_axis_name="s")

  @functools.partial(
      pl.kernel,
      out_type=jax.ShapeDtypeStruct((N, _HID), jnp.float32),
      mesh=mesh,
      scratch_types=[
          pltpu.VMEM((npw,), jnp.int32),
          pltpu.VMEM((_NBUF, _C, _HID), jnp.float32),
          pltpu.SemaphoreType.DMA((_NBUF,)),
          pltpu.SemaphoreType.DMA((_NBUF,)),
      ],
  )
  def body(tok_hbm, word_hbm, out_hbm, tok_v, rows_v, in_sem, out_sem):
    wid = lax.axis_index("s") * 2 + lax.axis_index("c")
    w0 = wid * npw
    pltpu.sync_copy(tok_hbm.at[pl.ds(w0, npw)], tok_v)

    def issue(c):
      b = lax.rem(c, _NBUF)
      pltpu.async_copy(word_hbm.at[tok_v.at[pl.ds(c * _C, _C)]], rows_v.at[b],
                       in_sem.at[b])

    issue(0)
    issue(1)

    def loop_body(i, carry):
      b = lax.rem(i, _NBUF)
      pltpu.make_async_copy(word_hbm.at[pl.ds(0, _C)], rows_v.at[b],
                            in_sem.at[b]).wait()
      pltpu.async_copy(rows_v.at[b], out_hbm.at[pl.ds(w0 + i * _C, _C)],
                       out_sem.at[b])

      @pl.when(i + 2 < per_w)
      def _():
        b2 = lax.rem(i + 2, _NBUF)

        @pl.when(i >= 1)
        def _():
          pltpu.make_async_copy(rows_v.at[b2], out_hbm.at[pl.ds(0, _C)],
                                out_sem.at[b2]).wait()

        issue(i + 2)

      return carry

    lax.fori_loop(0, per_w, loop_body, 0)
    for k in range(_NBUF):
      pltpu.make_async_copy(rows_v.at[k], out_hbm.at[pl.ds(0, _C)],
                            out_sem.at[k]).wait()

  return body(tokens_flat, word_emb)


def _tc_ln_body(g_ref, s_ref, p_ref, t_ref, gm_ref, bt_ref, o_ref):
  x = g_ref[...]
  seg = s_ref[...]
  t0 = t_ref[0][None, None, :]
  t1 = t_ref[1][None, None, :]
  seg_b = lax.broadcast_in_dim(seg.astype(jnp.float32), x.shape, (0, 1))
  x = x + p_ref[...][None, :, :] + (t0 + seg_b * (t1 - t0))
  mean = jnp.mean(x, axis=-1, keepdims=True)
  var = jnp.mean(x * x, axis=-1, keepdims=True) - mean * mean
  y = lax.rsqrt(var + 1e-12)
  o_ref[...] = (x - mean) * y * gm_ref[...] + bt_ref[...]


def _tc_ln(gathered, segments, pos_emb_l, type_emb, gamma, beta):
  B, L = segments.shape
  g3 = gathered.reshape(B, L, _HID)
  grid = (B // _BB,)
  return pl.pallas_call(
      _tc_ln_body,
      grid=grid,
      in_specs=[
          pl.BlockSpec((_BB, L, _HID), lambda i: (i, 0, 0)),
          pl.BlockSpec((_BB, L), lambda i: (i, 0)),
          pl.BlockSpec((L, _HID), lambda i: (0, 0)),
          pl.BlockSpec((2, _HID), lambda i: (0, 0)),
          pl.BlockSpec((_HID,), lambda i: (0,)),
          pl.BlockSpec((_HID,), lambda i: (0,)),
      ],
      out_specs=pl.BlockSpec((_BB, L, _HID), lambda i: (i, 0, 0)),
      out_shape=jax.ShapeDtypeStruct((B, L, _HID), jnp.float32),
  )(g3, segments, pos_emb_l, type_emb, gamma, beta)


def kernel(tokens, segments, word_emb, pos_emb, type_emb, gamma, beta):
  B, L = tokens.shape
  tok_flat = tokens.astype(jnp.int32).reshape(-1)
  seg = segments.astype(jnp.int32)
  pos_l = pos_emb[:L]
  halves = []
  h = B // 2
  for k in range(2):
    gath = _sc_gather(tok_flat[k * h * L:(k + 1) * h * L], word_emb)
    halves.append(_tc_ln(gath, seg[k * h:(k + 1) * h], pos_l, type_emb,
                         gamma, beta))
  return jnp.concatenate(halves, axis=0)


# speedup vs baseline: 1.7216x; 1.2387x over previous
import functools

import jax
import jax.numpy as jnp
from jax import lax
from jax.experimental import pallas as pl
from jax.experimental.pallas import tpu as pltpu
from jax.experimental.pallas import tpu_sc as plsc

_HID = 128
_C = 128
_NW = 32
_NBUF = 3
_BB = 16


def _sc_gather(tokens_flat, word_emb):
  N = tokens_flat.shape[0]
  per_w = N // _C // _NW
  npw = per_w * _C

  mesh = plsc.VectorSubcoreMesh(core_axis_name="c", subcore_axis_name="s")

  @functools.partial(
      pl.kernel,
      out_type=jax.ShapeDtypeStruct((N, _HID), jnp.float32),
      mesh=mesh,
      scratch_types=[
          pltpu.VMEM((npw,), jnp.int32),
          pltpu.VMEM((_NBUF, _C, _HID), jnp.float32),
          pltpu.SemaphoreType.DMA((_NBUF,)),
          pltpu.SemaphoreType.DMA((_NBUF,)),
      ],
  )
  def body(tok_hbm, word_hbm, out_hbm, tok_v, rows_v, in_sem, out_sem):
    wid = lax.axis_index("s") * 2 + lax.axis_index("c")
    w0 = wid * npw
    pltpu.sync_copy(tok_hbm.at[pl.ds(w0, npw)], tok_v)

    def issue(c):
      b = lax.rem(c, _NBUF)
      pltpu.async_copy(word_hbm.at[tok_v.at[pl.ds(c * _C, _C)]], rows_v.at[b],
                       in_sem.at[b])

    issue(0)
    issue(1)

    def loop_body(i, carry):
      b = lax.rem(i, _NBUF)
      pltpu.make_async_copy(word_hbm.at[pl.ds(0, _C)], rows_v.at[b],
                            in_sem.at[b]).wait()
      pltpu.async_copy(rows_v.at[b], out_hbm.at[pl.ds(w0 + i * _C, _C)],
                       out_sem.at[b])

      @pl.when(i + 2 < per_w)
      def _():
        b2 = lax.rem(i + 2, _NBUF)

        @pl.when(i >= 1)
        def _():
          pltpu.make_async_copy(rows_v.at[b2], out_hbm.at[pl.ds(0, _C)],
                                out_sem.at[b2]).wait()

        issue(i + 2)

      return carry

    lax.fori_loop(0, per_w, loop_body, 0)
    for k in range(_NBUF):
      pltpu.make_async_copy(rows_v.at[k], out_hbm.at[pl.ds(0, _C)],
                            out_sem.at[k]).wait()

  return body(tokens_flat, word_emb)


def _tc_ln_body(g_ref, s_ref, p_ref, t_ref, gm_ref, bt_ref, o_ref):
  x = g_ref[...]
  seg = s_ref[...]
  t0 = t_ref[0][None, None, :]
  t1 = t_ref[1][None, None, :]
  seg_b = lax.broadcast_in_dim(seg.astype(jnp.float32), x.shape, (0, 1))
  x = x + p_ref[...][None, :, :] + (t0 + seg_b * (t1 - t0))
  mean = jnp.mean(x, axis=-1, keepdims=True)
  var = jnp.mean(x * x, axis=-1, keepdims=True) - mean * mean
  y = lax.rsqrt(var + 1e-12)
  o_ref[...] = (x - mean) * y * gm_ref[...] + bt_ref[...]


def _tc_ln(gathered, segments, pos_emb_l, type_emb, gamma, beta):
  B, L = segments.shape
  g3 = gathered.reshape(B, L, _HID)
  grid = (B // _BB,)
  return pl.pallas_call(
      _tc_ln_body,
      grid=grid,
      in_specs=[
          pl.BlockSpec((_BB, L, _HID), lambda i: (i, 0, 0)),
          pl.BlockSpec((_BB, L), lambda i: (i, 0)),
          pl.BlockSpec((L, _HID), lambda i: (0, 0)),
          pl.BlockSpec((2, _HID), lambda i: (0, 0)),
          pl.BlockSpec((_HID,), lambda i: (0,)),
          pl.BlockSpec((_HID,), lambda i: (0,)),
      ],
      out_specs=pl.BlockSpec((_BB, L, _HID), lambda i: (i, 0, 0)),
      out_shape=jax.ShapeDtypeStruct((B, L, _HID), jnp.float32),
  )(g3, segments, pos_emb_l, type_emb, gamma, beta)


def kernel(tokens, segments, word_emb, pos_emb, type_emb, gamma, beta):
  B, L = tokens.shape
  tok_flat = tokens.astype(jnp.int32).reshape(-1)
  seg = segments.astype(jnp.int32)
  pos_l = pos_emb[:L]
  gath = _sc_gather(tok_flat, word_emb)
  return _tc_ln(gath, seg, pos_l, type_emb, gamma, beta)
